# Initial kernel scaffold; baseline (speedup 1.0000x reference)
import jax
import jax.numpy as jnp
from jax.experimental import pallas as pl


def kernel(embeddings):
    raise NotImplementedError("write your pallas kernel here")


# speedup vs baseline: 11.4149x; 11.4149x over previous
import functools

import jax
import jax.numpy as jnp
from jax import lax
from jax.experimental import pallas as pl
from jax.experimental.pallas import tpu as pltpu
from jax.experimental.pallas import tpu_sc as plsc

N = 4096
D = 256
K = 16
BLK = 256
NBLK = N // BLK

NC = 2
NS = 16
NW = NC * NS
RPW = N // NW
LANES = 16


def _topk_body(x_ref, idx_ref, dist_ref, sq_ref, dsum_ref):
    step = pl.program_id(0)
    X = x_ref[:, :]
    sq_full = jnp.sum(X * X, axis=1)
    Xb = x_ref[pl.ds(step * BLK, BLK), :]
    sq_b = jnp.sum(Xb * Xb, axis=1)
    S = lax.dot_general(Xb, X, (((1,), (1,)), ((), ())),
                        preferred_element_type=jnp.float32)
    d2 = sq_b[:, None] + sq_full[None, :] - 2.0 * S
    d2 = jnp.maximum(d2, 0.0)
    dist = jnp.sqrt(d2)

    col = lax.broadcasted_iota(jnp.int32, (BLK, N), 1)
    row = lax.broadcasted_iota(jnp.int32, (BLK, N), 0)
    inf = jnp.float32(jnp.inf)
    dist = jnp.where(col == row + step * BLK, inf, dist)

    ms = []
    ams = []
    dd = dist
    for _ in range(K):
        m = jnp.min(dd, axis=1)
        am = jnp.min(jnp.where(dd == m[:, None], col, N), axis=1)
        dd = jnp.where(col == am[:, None], inf, dd)
        ms.append(m)
        ams.append(am.astype(jnp.int32))
    dist_blk = jnp.stack(ms, axis=1)
    idx_blk = jnp.stack(ams, axis=1)

    idx_ref[:, :] = idx_blk
    dist_ref[:, :] = dist_blk
    sq_ref[0, :] = sq_b

    @pl.when(step == 0)
    def _():
        dsum_ref[0, 0] = 0.0

    dsum_ref[0, 0] += jnp.sum(dist_blk)


def _run_topk(x):
    return pl.pallas_call(
        _topk_body,
        grid=(NBLK,),
        in_specs=[pl.BlockSpec((N, D), lambda i: (0, 0))],
        out_specs=[
            pl.BlockSpec((BLK, K), lambda i: (i, 0)),
            pl.BlockSpec((BLK, K), lambda i: (i, 0)),
            pl.BlockSpec((1, BLK), lambda i: (0, i)),
            pl.BlockSpec(memory_space=pltpu.SMEM),
        ],
        out_shape=[
            jax.ShapeDtypeStruct((N, K), jnp.int32),
            jax.ShapeDtypeStruct((N, K), jnp.float32),
            jax.ShapeDtypeStruct((1, N), jnp.float32),
            jax.ShapeDtypeStruct((1, 1), jnp.float32),
        ],
    )(x)


def _sc_mesh():
    return plsc.VectorSubcoreMesh(core_axis_name="c", subcore_axis_name="s",
                                  num_cores=NC, num_subcores=NS)


def _edge_body(idx_hbm, dist_hbm, c_hbm,
               w_hbm, mm_hbm, degp_hbm,
               idx_v, d_v, w_v, mm_v, degp_v, c_v):
    wid = lax.axis_index("s") * NC + lax.axis_index("c")
    base = wid * RPW

    pltpu.sync_copy(idx_hbm, idx_v)
    pltpu.sync_copy(dist_hbm.at[pl.ds(base * K, RPW * K)], d_v)
    pltpu.sync_copy(c_hbm, c_v)

    cvec = c_v[...]
    lane = lax.iota(jnp.int32, LANES)

    def zero_chunk(t, _):
        degp_v[pl.ds(t * LANES, LANES)] = jnp.zeros((LANES,), jnp.float32)
        return 0

    lax.fori_loop(0, N // LANES, zero_chunk, 0)

    def chunk_body(ch, _):
        def row_body(t, acc):
            r = ch * LANES + t
            i = base + r
            jv = idx_v[pl.ds(i * K, K)]
            dd = d_v[pl.ds(r * K, K)]
            w = jnp.exp(-(dd * dd) * cvec)
            w_v[pl.ds(r * K, K)] = w
            jbase = jv * K
            mut = plsc.load_gather(idx_v, [jbase]) == i
            for l in range(1, K):
                coln = plsc.load_gather(idx_v, [jbase + l])
                mut = mut | (coln == i)
            mm_v[pl.ds(r * K, K)] = jnp.where(mut, 1.0, 2.0).astype(jnp.float32)
            plsc.addupdate_scatter(
                degp_v, [jv], jnp.where(mut, 0.0, w).astype(jnp.float32))
            return jnp.where(lane == t, acc + jnp.sum(w), acc)

        acc = lax.fori_loop(0, LANES, row_body,
                            jnp.zeros((LANES,), jnp.float32))
        s = pl.ds(base + ch * LANES, LANES)
        degp_v[s] = degp_v[s] + acc
        return 0

    lax.fori_loop(0, RPW // LANES, chunk_body, 0)

    pltpu.sync_copy(w_v, w_hbm.at[pl.ds(base * K, RPW * K)])
    pltpu.sync_copy(mm_v, mm_hbm.at[pl.ds(base * K, RPW * K)])
    pltpu.sync_copy(degp_v, degp_hbm.at[wid])


def _run_edges(idx, dists, cvec):
    f = pl.kernel(
        _edge_body,
        out_type=[
            jax.ShapeDtypeStruct((N * K,), jnp.float32),
            jax.ShapeDtypeStruct((N * K,), jnp.float32),
            jax.ShapeDtypeStruct((NW, N), jnp.float32),
        ],
        mesh=_sc_mesh(),
        compiler_params=pltpu.CompilerParams(needs_layout_passes=False),
        scratch_types=[
            pltpu.VMEM((N * K,), jnp.int32),
            pltpu.VMEM((RPW * K,), jnp.float32),
            pltpu.VMEM((RPW * K,), jnp.float32),
            pltpu.VMEM((RPW * K,), jnp.float32),
            pltpu.VMEM((N,), jnp.float32),
            pltpu.VMEM((LANES,), jnp.float32),
        ],
    )
    return f(idx, dists, cvec)


def _deg_body(degp_ref, sq_ref, dis_ref, diag_ref):
    deg = jnp.sum(degp_ref[:, :], axis=0)
    dis = 1.0 / jnp.sqrt(deg + 1e-10)
    dis_ref[0, :] = dis
    sq = sq_ref[0, :]
    diag_ref[0, 0] = jnp.sum(dis * dis * deg * sq)


def _run_deg(degp, sq):
    return pl.pallas_call(
        _deg_body,
        out_specs=[
            pl.BlockSpec((1, N), lambda: (0, 0)),
            pl.BlockSpec(memory_space=pltpu.SMEM),
        ],
        out_shape=[
            jax.ShapeDtypeStruct((1, N), jnp.float32),
            jax.ShapeDtypeStruct((1, 1), jnp.float32),
        ],
    )(degp, sq)


def _cross_body(idx_hbm, dist_hbm, w_hbm, mm_hbm, dis_hbm, sq_hbm,
                out_hbm,
                idx_v, d_v, w_v, mm_v, dis_v, sq_v, acc_v):
    wid = lax.axis_index("s") * NC + lax.axis_index("c")
    base = wid * RPW

    pltpu.sync_copy(idx_hbm.at[pl.ds(base * K, RPW * K)], idx_v)
    pltpu.sync_copy(dist_hbm.at[pl.ds(base * K, RPW * K)], d_v)
    pltpu.sync_copy(w_hbm.at[pl.ds(base * K, RPW * K)], w_v)
    pltpu.sync_copy(mm_hbm.at[pl.ds(base * K, RPW * K)], mm_v)
    pltpu.sync_copy(dis_hbm, dis_v)
    pltpu.sync_copy(sq_hbm, sq_v)

    def row_body(r, acc):
        i = base + r
        jv = idx_v[pl.ds(r * K, K)]
        dd = d_v[pl.ds(r * K, K)]
        w = w_v[pl.ds(r * K, K)]
        mm = mm_v[pl.ds(r * K, K)]
        isplat = jnp.full((LANES,), i, jnp.int32)
        dis_j = plsc.load_gather(dis_v, [jv])
        sq_j = plsc.load_gather(sq_v, [jv])
        dis_i = plsc.load_gather(dis_v, [isplat])
        sq_i = plsc.load_gather(sq_v, [isplat])
        g = 0.5 * (sq_i + sq_j - dd * dd)
        return acc + mm * w * dis_i * dis_j * g

    acc = lax.fori_loop(0, RPW, row_body, jnp.zeros((LANES,), jnp.float32))
    acc_v[...] = acc
    pltpu.sync_copy(acc_v, out_hbm.at[wid])


def _run_cross(idx, dists, w, mm, dis, sq):
    f = pl.kernel(
        _cross_body,
        out_type=jax.ShapeDtypeStruct((NW, LANES), jnp.float32),
        mesh=_sc_mesh(),
        compiler_params=pltpu.CompilerParams(needs_layout_passes=False),
        scratch_types=[
            pltpu.VMEM((RPW * K,), jnp.int32),
            pltpu.VMEM((RPW * K,), jnp.float32),
            pltpu.VMEM((RPW * K,), jnp.float32),
            pltpu.VMEM((RPW * K,), jnp.float32),
            pltpu.VMEM((N,), jnp.float32),
            pltpu.VMEM((N,), jnp.float32),
            pltpu.VMEM((LANES,), jnp.float32),
        ],
    )
    return f(idx, dists, w, mm, dis, sq)


def kernel(embeddings):
    idx, dists, sq2d, dsum = _run_topk(embeddings)
    idx_f = idx.reshape(N * K)
    dists_f = dists.reshape(N * K)
    sigma = dsum[0, 0] / jnp.float32(N * K)
    cvec = jnp.full((LANES,), 1.0, jnp.float32) / (2.0 * sigma * sigma)
    w, mm, degp = _run_edges(idx_f, dists_f, cvec)
    dis2d, diag = _run_deg(degp, sq2d)
    cross = _run_cross(idx_f, dists_f, w, mm, dis2d[0], sq2d[0])
    return diag[0, 0] - jnp.sum(cross)


# speedup vs baseline: 15.9407x; 1.3965x over previous
import functools

import jax
import jax.numpy as jnp
from jax import lax
from jax.experimental import pallas as pl
from jax.experimental.pallas import tpu as pltpu
from jax.experimental.pallas import tpu_sc as plsc

N = 4096
D = 256
K = 16
BLK = 256
NBLK = N // BLK

NC = 2
NS = 16
NW = NC * NS
RPW = N // NW
LANES = 16


def _topk_body(x_ref, idx_ref, dist_ref, sq_ref, dsum_ref):
    step = pl.program_id(0)
    X = x_ref[:, :]
    sq_full = jnp.sum(X * X, axis=1)
    Xb = x_ref[pl.ds(step * BLK, BLK), :]
    sq_b = jnp.sum(Xb * Xb, axis=1)
    S = lax.dot_general(Xb, X, (((1,), (1,)), ((), ())),
                        preferred_element_type=jnp.float32)
    d2 = sq_b[:, None] + sq_full[None, :] - 2.0 * S
    d2 = jnp.maximum(d2, 0.0)

    col = lax.broadcasted_iota(jnp.int32, (BLK, N), 1)
    row = lax.broadcasted_iota(jnp.int32, (BLK, N), 0)
    keys = (lax.bitcast_convert_type(d2, jnp.int32) &
            jnp.int32(-4096)) | col
    imax = jnp.int32(0x7FFFFFFF)
    keys = jnp.where(col == row + step * BLK, imax, keys)

    ms = []
    ams = []
    prev = jnp.full((BLK,), -1, jnp.int32)
    for _ in range(K):
        kmin = jnp.min(jnp.where(keys > prev[:, None], keys, imax),
                       axis=1)
        prev = kmin
        ams.append(kmin & jnp.int32(0xFFF))
        ms.append(jnp.sqrt(lax.bitcast_convert_type(
            kmin & jnp.int32(-4096), jnp.float32)))
    dist_blk = jnp.stack(ms, axis=1)
    idx_blk = jnp.stack(ams, axis=1)

    idx_ref[:, :] = idx_blk
    dist_ref[:, :] = dist_blk
    sq_ref[0, :] = sq_b

    @pl.when(step == 0)
    def _():
        dsum_ref[0, 0] = 0.0

    dsum_ref[0, 0] += jnp.sum(dist_blk)


def _run_topk(x):
    return pl.pallas_call(
        _topk_body,
        grid=(NBLK,),
        in_specs=[pl.BlockSpec((N, D), lambda i: (0, 0))],
        out_specs=[
            pl.BlockSpec((BLK, K), lambda i: (i, 0)),
            pl.BlockSpec((BLK, K), lambda i: (i, 0)),
            pl.BlockSpec((1, BLK), lambda i: (0, i)),
            pl.BlockSpec(memory_space=pltpu.SMEM),
        ],
        out_shape=[
            jax.ShapeDtypeStruct((N, K), jnp.int32),
            jax.ShapeDtypeStruct((N, K), jnp.float32),
            jax.ShapeDtypeStruct((1, N), jnp.float32),
            jax.ShapeDtypeStruct((1, 1), jnp.float32),
        ],
    )(x)


def _sc_mesh():
    return plsc.VectorSubcoreMesh(core_axis_name="c", subcore_axis_name="s",
                                  num_cores=NC, num_subcores=NS)


def _edge_body(idx_hbm, dist_hbm, c_hbm,
               w_hbm, mm_hbm, degp_hbm,
               idx_v, d_v, w_v, mm_v, degp_v, c_v):
    wid = lax.axis_index("s") * NC + lax.axis_index("c")
    base = wid * RPW

    pltpu.sync_copy(idx_hbm, idx_v)
    pltpu.sync_copy(dist_hbm.at[pl.ds(base * K, RPW * K)], d_v)
    pltpu.sync_copy(c_hbm, c_v)

    cvec = c_v[...]
    lane = lax.iota(jnp.int32, LANES)

    def zero_chunk(t, _):
        degp_v[pl.ds(t * LANES, LANES)] = jnp.zeros((LANES,), jnp.float32)
        return 0

    lax.fori_loop(0, N // LANES, zero_chunk, 0)

    def chunk_body(ch, _):
        def row_body(t, acc):
            r = ch * LANES + t
            i = base + r
            jv = idx_v[pl.ds(i * K, K)]
            dd = d_v[pl.ds(r * K, K)]
            w = jnp.exp(-(dd * dd) * cvec)
            w_v[pl.ds(r * K, K)] = w
            jbase = jv * K
            mut = plsc.load_gather(idx_v, [jbase]) == i
            for l in range(1, K):
                coln = plsc.load_gather(idx_v, [jbase + l])
                mut = mut | (coln == i)
            mm_v[pl.ds(r * K, K)] = jnp.where(mut, 1.0, 2.0).astype(jnp.float32)
            plsc.addupdate_scatter(
                degp_v, [jv], jnp.where(mut, 0.0, w).astype(jnp.float32))
            return jnp.where(lane == t, acc + jnp.sum(w), acc)

        acc = lax.fori_loop(0, LANES, row_body,
                            jnp.zeros((LANES,), jnp.float32))
        s = pl.ds(base + ch * LANES, LANES)
        degp_v[s] = degp_v[s] + acc
        return 0

    lax.fori_loop(0, RPW // LANES, chunk_body, 0)

    pltpu.sync_copy(w_v, w_hbm.at[pl.ds(base * K, RPW * K)])
    pltpu.sync_copy(mm_v, mm_hbm.at[pl.ds(base * K, RPW * K)])
    pltpu.sync_copy(degp_v, degp_hbm.at[wid])


def _run_edges(idx, dists, cvec):
    f = pl.kernel(
        _edge_body,
        out_type=[
            jax.ShapeDtypeStruct((N * K,), jnp.float32),
            jax.ShapeDtypeStruct((N * K,), jnp.float32),
            jax.ShapeDtypeStruct((NW, N), jnp.float32),
        ],
        mesh=_sc_mesh(),
        compiler_params=pltpu.CompilerParams(needs_layout_passes=False),
        scratch_types=[
            pltpu.VMEM((N * K,), jnp.int32),
            pltpu.VMEM((RPW * K,), jnp.float32),
            pltpu.VMEM((RPW * K,), jnp.float32),
            pltpu.VMEM((RPW * K,), jnp.float32),
            pltpu.VMEM((N,), jnp.float32),
            pltpu.VMEM((LANES,), jnp.float32),
        ],
    )
    return f(idx, dists, cvec)


def _deg_body(degp_ref, sq_ref, dis_ref, diag_ref):
    deg = jnp.sum(degp_ref[:, :], axis=0)
    dis = 1.0 / jnp.sqrt(deg + 1e-10)
    dis_ref[0, :] = dis
    sq = sq_ref[0, :]
    diag_ref[0, 0] = jnp.sum(dis * dis * deg * sq)


def _run_deg(degp, sq):
    return pl.pallas_call(
        _deg_body,
        out_specs=[
            pl.BlockSpec((1, N), lambda: (0, 0)),
            pl.BlockSpec(memory_space=pltpu.SMEM),
        ],
        out_shape=[
            jax.ShapeDtypeStruct((1, N), jnp.float32),
            jax.ShapeDtypeStruct((1, 1), jnp.float32),
        ],
    )(degp, sq)


def _cross_body(idx_hbm, dist_hbm, w_hbm, mm_hbm, dis_hbm, sq_hbm,
                out_hbm,
                idx_v, d_v, w_v, mm_v, dis_v, sq_v, acc_v):
    wid = lax.axis_index("s") * NC + lax.axis_index("c")
    base = wid * RPW

    pltpu.sync_copy(idx_hbm.at[pl.ds(base * K, RPW * K)], idx_v)
    pltpu.sync_copy(dist_hbm.at[pl.ds(base * K, RPW * K)], d_v)
    pltpu.sync_copy(w_hbm.at[pl.ds(base * K, RPW * K)], w_v)
    pltpu.sync_copy(mm_hbm.at[pl.ds(base * K, RPW * K)], mm_v)
    pltpu.sync_copy(dis_hbm, dis_v)
    pltpu.sync_copy(sq_hbm, sq_v)

    def row_body(r, acc):
        i = base + r
        jv = idx_v[pl.ds(r * K, K)]
        dd = d_v[pl.ds(r * K, K)]
        w = w_v[pl.ds(r * K, K)]
        mm = mm_v[pl.ds(r * K, K)]
        isplat = jnp.full((LANES,), i, jnp.int32)
        dis_j = plsc.load_gather(dis_v, [jv])
        sq_j = plsc.load_gather(sq_v, [jv])
        dis_i = plsc.load_gather(dis_v, [isplat])
        sq_i = plsc.load_gather(sq_v, [isplat])
        g = 0.5 * (sq_i + sq_j - dd * dd)
        return acc + mm * w * dis_i * dis_j * g

    acc = lax.fori_loop(0, RPW, row_body, jnp.zeros((LANES,), jnp.float32))
    acc_v[...] = acc
    pltpu.sync_copy(acc_v, out_hbm.at[wid])


def _run_cross(idx, dists, w, mm, dis, sq):
    f = pl.kernel(
        _cross_body,
        out_type=jax.ShapeDtypeStruct((NW, LANES), jnp.float32),
        mesh=_sc_mesh(),
        compiler_params=pltpu.CompilerParams(needs_layout_passes=False),
        scratch_types=[
            pltpu.VMEM((RPW * K,), jnp.int32),
            pltpu.VMEM((RPW * K,), jnp.float32),
            pltpu.VMEM((RPW * K,), jnp.float32),
            pltpu.VMEM((RPW * K,), jnp.float32),
            pltpu.VMEM((N,), jnp.float32),
            pltpu.VMEM((N,), jnp.float32),
            pltpu.VMEM((LANES,), jnp.float32),
        ],
    )
    return f(idx, dists, w, mm, dis, sq)


def kernel(embeddings):
    idx, dists, sq2d, dsum = _run_topk(embeddings)
    idx_f = idx.reshape(N * K)
    dists_f = dists.reshape(N * K)
    sigma = dsum[0, 0] / jnp.float32(N * K)
    cvec = jnp.full((LANES,), 1.0, jnp.float32) / (2.0 * sigma * sigma)
    w, mm, degp = _run_edges(idx_f, dists_f, cvec)
    dis2d, diag = _run_deg(degp, sq2d)
    cross = _run_cross(idx_f, dists_f, w, mm, dis2d[0], sq2d[0])
    return diag[0, 0] - jnp.sum(cross)


# speedup vs baseline: 18.1666x; 1.1396x over previous
import functools

import jax
import jax.numpy as jnp
from jax import lax
from jax.experimental import pallas as pl
from jax.experimental.pallas import tpu as pltpu
from jax.experimental.pallas import tpu_sc as plsc

N = 4096
D = 256
K = 16
BLK = 256
NBLK = N // BLK

NC = 2
NS = 16
NW = NC * NS
RPW = N // NW
LANES = 16


def _topk_body(x_ref, idx_ref, dist_ref, sq_ref, dsum_ref, sqs_ref):
    step = pl.program_id(0)
    X = x_ref[:, :]

    @pl.when(step == 0)
    def _():
        sqs_ref[0, :] = jnp.sum(X * X, axis=1)

    sq_full = sqs_ref[0, :]
    Xb = x_ref[pl.ds(step * BLK, BLK), :]
    sq_b = sqs_ref[0, pl.ds(step * BLK, BLK)]
    S = lax.dot_general(Xb, X, (((1,), (1,)), ((), ())),
                        preferred_element_type=jnp.float32)
    d2 = sq_b[:, None] + sq_full[None, :] - 2.0 * S
    d2 = jnp.maximum(d2, 0.0)

    col = lax.broadcasted_iota(jnp.int32, (BLK, N), 1)
    row = lax.broadcasted_iota(jnp.int32, (BLK, N), 0)
    keys = (lax.bitcast_convert_type(d2, jnp.int32) &
            jnp.int32(-4096)) | col
    imax = jnp.int32(0x7FFFFFFF)
    keys = jnp.where(col == row + step * BLK, imax, keys)

    ms = []
    ams = []
    kmin = jnp.min(keys, axis=1)
    for k in range(K):
        if k > 0:
            base2 = kmin - jnp.int32(2147483647)
            v = keys - base2[:, None]
            kmin = jnp.min(v, axis=1) + base2
        ams.append(kmin & jnp.int32(0xFFF))
        ms.append(jnp.sqrt(lax.bitcast_convert_type(
            kmin & jnp.int32(-4096), jnp.float32)))
    dist_blk = jnp.stack(ms, axis=1)
    idx_blk = jnp.stack(ams, axis=1)

    idx_ref[:, :] = idx_blk
    dist_ref[:, :] = dist_blk
    sq_ref[0, :] = sq_b

    @pl.when(step == 0)
    def _():
        dsum_ref[0, 0] = 0.0

    dsum_ref[0, 0] += jnp.sum(dist_blk)


def _run_topk(x):
    return pl.pallas_call(
        _topk_body,
        grid=(NBLK,),
        in_specs=[pl.BlockSpec((N, D), lambda i: (0, 0))],
        out_specs=[
            pl.BlockSpec((BLK, K), lambda i: (i, 0)),
            pl.BlockSpec((BLK, K), lambda i: (i, 0)),
            pl.BlockSpec((1, BLK), lambda i: (0, i)),
            pl.BlockSpec(memory_space=pltpu.SMEM),
        ],
        out_shape=[
            jax.ShapeDtypeStruct((N, K), jnp.int32),
            jax.ShapeDtypeStruct((N, K), jnp.float32),
            jax.ShapeDtypeStruct((1, N), jnp.float32),
            jax.ShapeDtypeStruct((1, 1), jnp.float32),
        ],
        scratch_shapes=[pltpu.VMEM((1, N), jnp.float32)],
    )(x)


def _sc_mesh():
    return plsc.VectorSubcoreMesh(core_axis_name="c", subcore_axis_name="s",
                                  num_cores=NC, num_subcores=NS)


def _edge_body(idx_hbm, dist_hbm, c_hbm,
               w_hbm, mm_hbm, degp_hbm,
               idx_v, d_v, w_v, mm_v, degp_v, c_v):
    wid = lax.axis_index("s") * NC + lax.axis_index("c")
    base = wid * RPW

    pltpu.sync_copy(idx_hbm, idx_v)
    pltpu.sync_copy(dist_hbm.at[pl.ds(base * K, RPW * K)], d_v)
    pltpu.sync_copy(c_hbm, c_v)

    cvec = c_v[...]
    lane = lax.iota(jnp.int32, LANES)

    def zero_chunk(t, _):
        degp_v[pl.ds(t * LANES, LANES)] = jnp.zeros((LANES,), jnp.float32)
        return 0

    lax.fori_loop(0, N // LANES, zero_chunk, 0)

    def chunk_body(ch, _):
        def row_body(t, acc):
            r = ch * LANES + t
            i = base + r
            jv = idx_v[pl.ds(i * K, K)]
            dd = d_v[pl.ds(r * K, K)]
            w = jnp.exp(-(dd * dd) * cvec)
            w_v[pl.ds(r * K, K)] = w
            jbase = jv * K
            mut = plsc.load_gather(idx_v, [jbase]) == i
            for l in range(1, K):
                coln = plsc.load_gather(idx_v, [jbase + l])
                mut = mut | (coln == i)
            mm_v[pl.ds(r * K, K)] = jnp.where(mut, 1.0, 2.0).astype(jnp.float32)
            plsc.addupdate_scatter(
                degp_v, [jv], jnp.where(mut, 0.0, w).astype(jnp.float32))
            return jnp.where(lane == t, acc + jnp.sum(w), acc)

        acc = lax.fori_loop(0, LANES, row_body,
                            jnp.zeros((LANES,), jnp.float32))
        s = pl.ds(base + ch * LANES, LANES)
        degp_v[s] = degp_v[s] + acc
        return 0

    lax.fori_loop(0, RPW // LANES, chunk_body, 0)

    pltpu.sync_copy(w_v, w_hbm.at[pl.ds(base * K, RPW * K)])
    pltpu.sync_copy(mm_v, mm_hbm.at[pl.ds(base * K, RPW * K)])
    pltpu.sync_copy(degp_v, degp_hbm.at[wid])


def _run_edges(idx, dists, cvec):
    f = pl.kernel(
        _edge_body,
        out_type=[
            jax.ShapeDtypeStruct((N * K,), jnp.float32),
            jax.ShapeDtypeStruct((N * K,), jnp.float32),
            jax.ShapeDtypeStruct((NW, N), jnp.float32),
        ],
        mesh=_sc_mesh(),
        compiler_params=pltpu.CompilerParams(needs_layout_passes=False),
        scratch_types=[
            pltpu.VMEM((N * K,), jnp.int32),
            pltpu.VMEM((RPW * K,), jnp.float32),
            pltpu.VMEM((RPW * K,), jnp.float32),
            pltpu.VMEM((RPW * K,), jnp.float32),
            pltpu.VMEM((N,), jnp.float32),
            pltpu.VMEM((LANES,), jnp.float32),
        ],
    )
    return f(idx, dists, cvec)


def _deg_body(degp_ref, sq_ref, dis_ref, diag_ref):
    deg = jnp.sum(degp_ref[:, :], axis=0)
    dis = 1.0 / jnp.sqrt(deg + 1e-10)
    dis_ref[0, :] = dis
    sq = sq_ref[0, :]
    diag_ref[0, 0] = jnp.sum(dis * dis * deg * sq)


def _run_deg(degp, sq):
    return pl.pallas_call(
        _deg_body,
        out_specs=[
            pl.BlockSpec((1, N), lambda: (0, 0)),
            pl.BlockSpec(memory_space=pltpu.SMEM),
        ],
        out_shape=[
            jax.ShapeDtypeStruct((1, N), jnp.float32),
            jax.ShapeDtypeStruct((1, 1), jnp.float32),
        ],
    )(degp, sq)


def _cross_body(idx_hbm, dist_hbm, w_hbm, mm_hbm, dis_hbm, sq_hbm,
                out_hbm,
                idx_v, d_v, w_v, mm_v, dis_v, sq_v, acc_v):
    wid = lax.axis_index("s") * NC + lax.axis_index("c")
    base = wid * RPW

    pltpu.sync_copy(idx_hbm.at[pl.ds(base * K, RPW * K)], idx_v)
    pltpu.sync_copy(dist_hbm.at[pl.ds(base * K, RPW * K)], d_v)
    pltpu.sync_copy(w_hbm.at[pl.ds(base * K, RPW * K)], w_v)
    pltpu.sync_copy(mm_hbm.at[pl.ds(base * K, RPW * K)], mm_v)
    pltpu.sync_copy(dis_hbm, dis_v)
    pltpu.sync_copy(sq_hbm, sq_v)

    def row_body(r, acc):
        i = base + r
        jv = idx_v[pl.ds(r * K, K)]
        dd = d_v[pl.ds(r * K, K)]
        w = w_v[pl.ds(r * K, K)]
        mm = mm_v[pl.ds(r * K, K)]
        isplat = jnp.full((LANES,), i, jnp.int32)
        dis_j = plsc.load_gather(dis_v, [jv])
        sq_j = plsc.load_gather(sq_v, [jv])
        dis_i = plsc.load_gather(dis_v, [isplat])
        sq_i = plsc.load_gather(sq_v, [isplat])
        g = 0.5 * (sq_i + sq_j - dd * dd)
        return acc + mm * w * dis_i * dis_j * g

    acc = lax.fori_loop(0, RPW, row_body, jnp.zeros((LANES,), jnp.float32))
    acc_v[...] = acc
    pltpu.sync_copy(acc_v, out_hbm.at[wid])


def _run_cross(idx, dists, w, mm, dis, sq):
    f = pl.kernel(
        _cross_body,
        out_type=jax.ShapeDtypeStruct((NW, LANES), jnp.float32),
        mesh=_sc_mesh(),
        compiler_params=pltpu.CompilerParams(needs_layout_passes=False),
        scratch_types=[
            pltpu.VMEM((RPW * K,), jnp.int32),
            pltpu.VMEM((RPW * K,), jnp.float32),
            pltpu.VMEM((RPW * K,), jnp.float32),
            pltpu.VMEM((RPW * K,), jnp.float32),
            pltpu.VMEM((N,), jnp.float32),
            pltpu.VMEM((N,), jnp.float32),
            pltpu.VMEM((LANES,), jnp.float32),
        ],
    )
    return f(idx, dists, w, mm, dis, sq)


def kernel(embeddings):
    idx, dists, sq2d, dsum = _run_topk(embeddings)
    idx_f = idx.reshape(N * K)
    dists_f = dists.reshape(N * K)
    sigma = dsum[0, 0] / jnp.float32(N * K)
    cvec = jnp.full((LANES,), 1.0, jnp.float32) / (2.0 * sigma * sigma)
    w, mm, degp = _run_edges(idx_f, dists_f, cvec)
    dis2d, diag = _run_deg(degp, sq2d)
    cross = _run_cross(idx_f, dists_f, w, mm, dis2d[0], sq2d[0])
    return diag[0, 0] - jnp.sum(cross)


# speedup vs baseline: 20.0371x; 1.1030x over previous
import functools

import jax
import jax.numpy as jnp
from jax import lax
from jax.experimental import pallas as pl
from jax.experimental.pallas import tpu as pltpu
from jax.experimental.pallas import tpu_sc as plsc

N = 4096
D = 256
K = 16
BLK = 512
NBLK = N // BLK

NC = 2
NS = 16
NW = NC * NS
RPW = N // NW
LANES = 16


def _topk_body(x_ref, idx_ref, dist_ref, sq_ref, dsum_ref, sqs_ref):
    step = pl.program_id(0)
    X = x_ref[:, :]

    @pl.when(step == 0)
    def _():
        sqs_ref[0, :] = jnp.sum(X * X, axis=1)

    sq_full = sqs_ref[0, :]
    Xb = x_ref[pl.ds(step * BLK, BLK), :]
    sq_b = sqs_ref[0, pl.ds(step * BLK, BLK)]
    S = lax.dot_general(Xb, X, (((1,), (1,)), ((), ())),
                        preferred_element_type=jnp.float32)
    d2 = sq_b[:, None] + sq_full[None, :] - 2.0 * S
    d2 = jnp.maximum(d2, 0.0)

    col = lax.broadcasted_iota(jnp.int32, (BLK, N), 1)
    row = lax.broadcasted_iota(jnp.int32, (BLK, N), 0)
    keys = (lax.bitcast_convert_type(d2, jnp.int32) &
            jnp.int32(-4096)) | col
    imax = jnp.int32(0x7FFFFFFF)
    keys = jnp.where(col == row + step * BLK, imax, keys)

    ms = []
    ams = []
    kmin = jnp.min(keys, axis=1)
    for k in range(K):
        if k > 0:
            base2 = kmin - jnp.int32(2147483647)
            v = keys - base2[:, None]
            kmin = jnp.min(v, axis=1) + base2
        ams.append(kmin & jnp.int32(0xFFF))
        ms.append(jnp.sqrt(lax.bitcast_convert_type(
            kmin & jnp.int32(-4096), jnp.float32)))
    dist_blk = jnp.stack(ms, axis=1)
    idx_blk = jnp.stack(ams, axis=1)

    idx_ref[:, :] = idx_blk
    dist_ref[:, :] = dist_blk
    sq_ref[0, :] = sq_b

    @pl.when(step == 0)
    def _():
        dsum_ref[0, 0] = 0.0

    dsum_ref[0, 0] += jnp.sum(dist_blk)


def _run_topk(x):
    return pl.pallas_call(
        _topk_body,
        grid=(NBLK,),
        in_specs=[pl.BlockSpec((N, D), lambda i: (0, 0))],
        out_specs=[
            pl.BlockSpec((BLK, K), lambda i: (i, 0)),
            pl.BlockSpec((BLK, K), lambda i: (i, 0)),
            pl.BlockSpec((1, BLK), lambda i: (0, i)),
            pl.BlockSpec(memory_space=pltpu.SMEM),
        ],
        out_shape=[
            jax.ShapeDtypeStruct((N, K), jnp.int32),
            jax.ShapeDtypeStruct((N, K), jnp.float32),
            jax.ShapeDtypeStruct((1, N), jnp.float32),
            jax.ShapeDtypeStruct((1, 1), jnp.float32),
        ],
        scratch_shapes=[pltpu.VMEM((1, N), jnp.float32)],
    )(x)


def _sc_mesh():
    return plsc.VectorSubcoreMesh(core_axis_name="c", subcore_axis_name="s",
                                  num_cores=NC, num_subcores=NS)


def _edge_body(idx_hbm, dist_hbm, c_hbm,
               w_hbm, mm_hbm, degp_hbm,
               idx_v, d_v, w_v, mm_v, degp_v, c_v):
    wid = lax.axis_index("s") * NC + lax.axis_index("c")
    base = wid * RPW

    pltpu.sync_copy(idx_hbm, idx_v)
    pltpu.sync_copy(dist_hbm.at[pl.ds(base * K, RPW * K)], d_v)
    pltpu.sync_copy(c_hbm, c_v)

    cvec = c_v[...]
    lane = lax.iota(jnp.int32, LANES)

    def zero_chunk(t, _):
        degp_v[pl.ds(t * LANES, LANES)] = jnp.zeros((LANES,), jnp.float32)
        return 0

    lax.fori_loop(0, N // LANES, zero_chunk, 0)

    def chunk_body(ch, _):
        def row_body(t, acc):
            r = ch * LANES + t
            i = base + r
            jv = idx_v[pl.ds(i * K, K)]
            dd = d_v[pl.ds(r * K, K)]
            w = jnp.exp(-(dd * dd) * cvec)
            w_v[pl.ds(r * K, K)] = w
            jbase = jv * K
            mut = plsc.load_gather(idx_v, [jbase]) == i
            for l in range(1, K):
                coln = plsc.load_gather(idx_v, [jbase + l])
                mut = mut | (coln == i)
            mm_v[pl.ds(r * K, K)] = jnp.where(mut, 1.0, 2.0).astype(jnp.float32)
            plsc.addupdate_scatter(
                degp_v, [jv], jnp.where(mut, 0.0, w).astype(jnp.float32))
            return jnp.where(lane == t, acc + jnp.sum(w), acc)

        acc = lax.fori_loop(0, LANES, row_body,
                            jnp.zeros((LANES,), jnp.float32))
        s = pl.ds(base + ch * LANES, LANES)
        degp_v[s] = degp_v[s] + acc
        return 0

    lax.fori_loop(0, RPW // LANES, chunk_body, 0)

    pltpu.sync_copy(w_v, w_hbm.at[pl.ds(base * K, RPW * K)])
    pltpu.sync_copy(mm_v, mm_hbm.at[pl.ds(base * K, RPW * K)])
    pltpu.sync_copy(degp_v, degp_hbm.at[wid])


def _run_edges(idx, dists, cvec):
    f = pl.kernel(
        _edge_body,
        out_type=[
            jax.ShapeDtypeStruct((N * K,), jnp.float32),
            jax.ShapeDtypeStruct((N * K,), jnp.float32),
            jax.ShapeDtypeStruct((NW, N), jnp.float32),
        ],
        mesh=_sc_mesh(),
        compiler_params=pltpu.CompilerParams(needs_layout_passes=False),
        scratch_types=[
            pltpu.VMEM((N * K,), jnp.int32),
            pltpu.VMEM((RPW * K,), jnp.float32),
            pltpu.VMEM((RPW * K,), jnp.float32),
            pltpu.VMEM((RPW * K,), jnp.float32),
            pltpu.VMEM((N,), jnp.float32),
            pltpu.VMEM((LANES,), jnp.float32),
        ],
    )
    return f(idx, dists, cvec)


def _deg_body(degp_ref, sq_ref, dis_ref, diag_ref):
    deg = jnp.sum(degp_ref[:, :], axis=0)
    dis = 1.0 / jnp.sqrt(deg + 1e-10)
    dis_ref[0, :] = dis
    sq = sq_ref[0, :]
    diag_ref[0, 0] = jnp.sum(dis * dis * deg * sq)


def _run_deg(degp, sq):
    return pl.pallas_call(
        _deg_body,
        out_specs=[
            pl.BlockSpec((1, N), lambda: (0, 0)),
            pl.BlockSpec(memory_space=pltpu.SMEM),
        ],
        out_shape=[
            jax.ShapeDtypeStruct((1, N), jnp.float32),
            jax.ShapeDtypeStruct((1, 1), jnp.float32),
        ],
    )(degp, sq)


def _cross_body(idx_hbm, dist_hbm, w_hbm, mm_hbm, dis_hbm, sq_hbm,
                out_hbm,
                idx_v, d_v, w_v, mm_v, dis_v, sq_v, acc_v):
    wid = lax.axis_index("s") * NC + lax.axis_index("c")
    base = wid * RPW

    pltpu.sync_copy(idx_hbm.at[pl.ds(base * K, RPW * K)], idx_v)
    pltpu.sync_copy(dist_hbm.at[pl.ds(base * K, RPW * K)], d_v)
    pltpu.sync_copy(w_hbm.at[pl.ds(base * K, RPW * K)], w_v)
    pltpu.sync_copy(mm_hbm.at[pl.ds(base * K, RPW * K)], mm_v)
    pltpu.sync_copy(dis_hbm, dis_v)
    pltpu.sync_copy(sq_hbm, sq_v)

    def row_body(r, acc):
        i = base + r
        jv = idx_v[pl.ds(r * K, K)]
        dd = d_v[pl.ds(r * K, K)]
        w = w_v[pl.ds(r * K, K)]
        mm = mm_v[pl.ds(r * K, K)]
        isplat = jnp.full((LANES,), i, jnp.int32)
        dis_j = plsc.load_gather(dis_v, [jv])
        sq_j = plsc.load_gather(sq_v, [jv])
        dis_i = plsc.load_gather(dis_v, [isplat])
        sq_i = plsc.load_gather(sq_v, [isplat])
        g = 0.5 * (sq_i + sq_j - dd * dd)
        return acc + mm * w * dis_i * dis_j * g

    acc = lax.fori_loop(0, RPW, row_body, jnp.zeros((LANES,), jnp.float32))
    acc_v[...] = acc
    pltpu.sync_copy(acc_v, out_hbm.at[wid])


def _run_cross(idx, dists, w, mm, dis, sq):
    f = pl.kernel(
        _cross_body,
        out_type=jax.ShapeDtypeStruct((NW, LANES), jnp.float32),
        mesh=_sc_mesh(),
        compiler_params=pltpu.CompilerParams(needs_layout_passes=False),
        scratch_types=[
            pltpu.VMEM((RPW * K,), jnp.int32),
            pltpu.VMEM((RPW * K,), jnp.float32),
            pltpu.VMEM((RPW * K,), jnp.float32),
            pltpu.VMEM((RPW * K,), jnp.float32),
            pltpu.VMEM((N,), jnp.float32),
            pltpu.VMEM((N,), jnp.float32),
            pltpu.VMEM((LANES,), jnp.float32),
        ],
    )
    return f(idx, dists, w, mm, dis, sq)


def kernel(embeddings):
    idx, dists, sq2d, dsum = _run_topk(embeddings)
    idx_f = idx.reshape(N * K)
    dists_f = dists.reshape(N * K)
    sigma = dsum[0, 0] / jnp.float32(N * K)
    cvec = jnp.full((LANES,), 1.0, jnp.float32) / (2.0 * sigma * sigma)
    w, mm, degp = _run_edges(idx_f, dists_f, cvec)
    dis2d, diag = _run_deg(degp, sq2d)
    cross = _run_cross(idx_f, dists_f, w, mm, dis2d[0], sq2d[0])
    return diag[0, 0] - jnp.sum(cross)
